# baseline (speedup 1.0000x reference)
import jax
import jax.numpy as jnp
from jax.experimental import pallas as pl
from jax.experimental.pallas import tpu as pltpu

N = 10000
E = 320000
D = 128
G = 64


def _mm_body(x_ref, w_ref, o_ref):
    o_ref[...] = jnp.dot(x_ref[...], w_ref[...],
                         preferred_element_type=jnp.float32)


def _matmul(x, wcat):
    blk = 2000
    return pl.pallas_call(
        _mm_body,
        grid=(N // blk,),
        in_specs=[
            pl.BlockSpec((blk, D), lambda i: (i, 0)),
            pl.BlockSpec((D, wcat.shape[1]), lambda i: (0, 0)),
        ],
        out_specs=pl.BlockSpec((blk, wcat.shape[1]), lambda i: (i, 0)),
        out_shape=jax.ShapeDtypeStruct((N, wcat.shape[1]), jnp.float32),
    )(x, wcat)


def _layer(x, src, dst, W, a_src, a_dst, b):
    acat = jnp.zeros((D, 256), jnp.float32)
    acat = acat.at[:, 0].set(W @ a_src).at[:, 128].set(W @ a_dst)
    wcat = jnp.concatenate([W, acat], axis=1)
    out = _matmul(x, wcat)
    h = out[:, :D]
    s = out[:, D]
    t = out[:, D + 128]
    e = s[src] + t[dst]
    e = jnp.where(e > 0, e, 0.2 * e)
    ex = jnp.exp(e)
    denom = jax.ops.segment_sum(ex, dst, num_segments=N)
    alpha = ex / (denom[dst] + 1e-16)
    out = jax.ops.segment_sum(h[src] * alpha[:, None], dst, num_segments=N)
    return out + b


def kernel(x, edge_index, batch, W1, a_src1, a_dst1, b1, W2, a_src2, a_dst2, b2):
    src = edge_index[0]
    dst = edge_index[1]
    h = jax.nn.relu(_layer(x, src, dst, W1, a_src1, a_dst1, b1))
    h = jax.nn.relu(_layer(h, src, dst, W2, a_src2, a_dst2, b2))
    sums = jax.ops.segment_sum(h, batch, num_segments=G)
    cnt = jax.ops.segment_sum(jnp.ones((N,), h.dtype), batch, num_segments=G)
    return sums / jnp.maximum(cnt, 1.0)[:, None]


# speedup vs baseline: 13.3206x; 13.3206x over previous
import functools

import jax
import jax.numpy as jnp
from jax import lax
from jax.experimental import pallas as pl
from jax.experimental.pallas import tpu as pltpu
from jax.experimental.pallas import tpu_sc as plsc

N = 10000
E = 320000
D = 128
HD = D // 2
G = 64

NC = 2
NS = 16
NP = 10240
NPR = NP // D
EPT = E // (NC * NS)
EPT2 = E // NS
B = 80
NB2 = EPT2 // B
SLICE = NP // NS


@functools.cache
def _mesh():
    return plsc.VectorSubcoreMesh(core_axis_name="c", subcore_axis_name="s",
                                  num_cores=NC, num_subcores=NS)


def _mm1_body(x_ref, w_ref, a_ref, h_ref, st_ref):
    xv = x_ref[...]
    h_ref[...] = jnp.dot(xv, w_ref[...], preferred_element_type=jnp.float32)
    st_ref[...] = jnp.dot(xv, a_ref[...], preferred_element_type=jnp.float32)


def _mm2_body(p0_ref, p1_ref, b_ref, w_ref, a_ref, h_ref, st_ref):
    xv = jnp.concatenate([p0_ref[...], p1_ref[...]], axis=1)
    xv = jnp.maximum(xv + b_ref[...], 0.0)
    h_ref[...] = jnp.dot(xv, w_ref[...], preferred_element_type=jnp.float32)
    st_ref[...] = jnp.dot(xv, a_ref[...], preferred_element_type=jnp.float32)


_MM_BLK = 2000


def _tc_layer1(x, W, acat):
    return pl.pallas_call(
        _mm1_body,
        grid=(N // _MM_BLK,),
        in_specs=[
            pl.BlockSpec((_MM_BLK, D), lambda i: (i, 0)),
            pl.BlockSpec((D, D), lambda i: (0, 0)),
            pl.BlockSpec((D, 8), lambda i: (0, 0)),
        ],
        out_specs=[
            pl.BlockSpec((_MM_BLK, D), lambda i: (i, 0)),
            pl.BlockSpec((_MM_BLK, 8), lambda i: (i, 0)),
        ],
        out_shape=[
            jax.ShapeDtypeStruct((N, D), jnp.float32),
            jax.ShapeDtypeStruct((N, 8), jnp.float32),
        ],
    )(x, W, acat)


def _tc_layer2(p0, p1, b, W, acat):
    return pl.pallas_call(
        _mm2_body,
        grid=(N // _MM_BLK,),
        in_specs=[
            pl.BlockSpec((_MM_BLK, HD), lambda i: (i, 0)),
            pl.BlockSpec((_MM_BLK, HD), lambda i: (i, 0)),
            pl.BlockSpec((1, D), lambda i: (0, 0)),
            pl.BlockSpec((D, D), lambda i: (0, 0)),
            pl.BlockSpec((D, 8), lambda i: (0, 0)),
        ],
        out_specs=[
            pl.BlockSpec((_MM_BLK, D), lambda i: (i, 0)),
            pl.BlockSpec((_MM_BLK, 8), lambda i: (i, 0)),
        ],
        out_shape=[
            jax.ShapeDtypeStruct((N, D), jnp.float32),
            jax.ShapeDtypeStruct((N, 8), jnp.float32),
        ],
    )(p0, p1, b, W, acat)


def _pool_body(p0_ref, p1_ref, b_ref, batch_ref, sum_ref, cnt_ref):
    i = pl.program_id(0)
    xv = jnp.concatenate([p0_ref[...], p1_ref[...]], axis=1)
    xv = jnp.maximum(xv + b_ref[...], 0.0)
    bt = batch_ref[...].reshape(1, _MM_BLK)
    oh = (lax.broadcasted_iota(jnp.int32, (G, _MM_BLK), 0) == bt)
    oh = oh.astype(jnp.float32)
    psum = jnp.dot(oh, xv, preferred_element_type=jnp.float32)
    pcnt = jnp.sum(oh, axis=1, keepdims=True)

    @pl.when(i == 0)
    def _():
        sum_ref[...] = jnp.zeros_like(sum_ref)
        cnt_ref[...] = jnp.zeros_like(cnt_ref)

    sum_ref[...] += psum
    cnt_ref[...] += jnp.broadcast_to(pcnt, (G, D))

    @pl.when(i == pl.num_programs(0) - 1)
    def _():
        sum_ref[...] = sum_ref[...] / jnp.maximum(cnt_ref[...], 1.0)


def _tc_pool(p0, p1, b, batch3d):
    out, _ = pl.pallas_call(
        _pool_body,
        grid=(N // _MM_BLK,),
        in_specs=[
            pl.BlockSpec((_MM_BLK, HD), lambda i: (i, 0)),
            pl.BlockSpec((_MM_BLK, HD), lambda i: (i, 0)),
            pl.BlockSpec((1, D), lambda i: (0, 0)),
            pl.BlockSpec((1, 1, _MM_BLK), lambda i: (i, 0, 0)),
        ],
        out_specs=[
            pl.BlockSpec((G, D), lambda i: (0, 0)),
            pl.BlockSpec((G, D), lambda i: (0, 0)),
        ],
        out_shape=[
            jax.ShapeDtypeStruct((G, D), jnp.float32),
            jax.ShapeDtypeStruct((G, D), jnp.float32),
        ],
    )(p0, p1, b, batch3d)
    return out


@functools.cache
def _sc_edge_pass_kernel():
  return functools.partial(
    pl.kernel,
    out_type=jax.ShapeDtypeStruct((NC, NPR, D), jnp.float32),
    mesh=_mesh(),
    scratch_types=[
        pltpu.VMEM((EPT,), jnp.int32),
        pltpu.VMEM((EPT,), jnp.int32),
        pltpu.VMEM((N,), jnp.float32),
        pltpu.VMEM((N,), jnp.float32),
        pltpu.VMEM((NPR, D), jnp.float32),
        pltpu.VMEM((NPR,), jnp.int32),
        pltpu.VMEM((5, D), jnp.float32),
        pltpu.VMEM_SHARED((NPR, D), jnp.float32),
    ],
    compiler_params=pltpu.CompilerParams(needs_layout_passes=False),
  )(_sc_edge_pass_body)


def _sc_edge_pass_body(srce, dste, s, t, dpart, src_v, dst_v, s_v, t_v,
                       den_v, rix_v, z5_v, stage):
    c = lax.axis_index("c")
    sid = lax.axis_index("s")
    base = (c * NS + sid) * EPT

    pltpu.sync_copy(srce.at[pl.ds(base, EPT)], src_v)
    pltpu.sync_copy(dste.at[pl.ds(base, EPT)], dst_v)
    pltpu.sync_copy(s, s_v)
    pltpu.sync_copy(t, t_v)

    zero16 = jnp.zeros((16,), jnp.float32)
    for r in range(5):
        for f in range(D // 16):
            z5_v[r, pl.ds(f * 16, 16)] = zero16

    def _zero_den(j, carry):
        for f in range(D // 16):
            den_v[j, pl.ds(f * 16, 16)] = zero16
        return carry

    lax.fori_loop(0, NPR, _zero_den, 0)

    for k in range(NPR // 16):
        rix_v[pl.ds(k * 16, 16)] = lax.iota(jnp.int32, 16) + k * 16

    pltpu.sync_copy(z5_v, stage.at[pl.ds(sid * 5, 5)])

    def _edge(j, carry):
        s16 = src_v[pl.ds(j * 16, 16)]
        d16 = dst_v[pl.ds(j * 16, 16)]
        ev = plsc.load_gather(s_v, [s16]) + plsc.load_gather(t_v, [d16])
        ev = jnp.where(ev > 0.0, ev, 0.2 * ev)
        exv = jnp.exp(ev)
        row16 = lax.shift_right_logical(d16, 7)
        col16 = lax.bitwise_and(d16, 127)
        plsc.addupdate_scatter(den_v, [row16, col16], exv)
        return carry

    lax.fori_loop(0, EPT // 16, _edge, 0)

    plsc.subcore_barrier()
    pltpu.sync_copy(den_v, stage.at[rix_v], add=True)
    plsc.subcore_barrier()

    @pl.when(sid < NPR // 8)
    def _():
        pltpu.sync_copy(stage.at[pl.ds(sid * 8, 8)],
                        dpart.at[c, pl.ds(sid * 8, 8)])


@functools.cache
def _sc_aggregate_kernel():
  return functools.partial(
    pl.kernel,
    out_type=jax.ShapeDtypeStruct((NC, NP, HD), jnp.float32),
    mesh=_mesh(),
    scratch_types=[
        pltpu.VMEM((EPT2,), jnp.int32),
        pltpu.VMEM((EPT2,), jnp.int32),
        pltpu.VMEM((N,), jnp.float32),
        pltpu.VMEM((N,), jnp.float32),
        pltpu.VMEM((NPR, D), jnp.float32),
        pltpu.VMEM((NPR, D), jnp.float32),
        pltpu.VMEM((B, HD), jnp.float32),
        pltpu.VMEM((B,), jnp.float32),
        pltpu.VMEM((B,), jnp.int32),
        pltpu.SemaphoreType.DMA,
        pltpu.VMEM_SHARED((NP, HD), jnp.float32),
    ],
    compiler_params=pltpu.CompilerParams(needs_layout_passes=False,
                                         use_tc_tiling_on_sc=False),
  )(_sc_aggregate_body)


def _sc_aggregate_body(srce, dste, s, t, dpart, hcat, opart, src_v, dst_v,
                       s_v, t_v, den_v, tmp_v, rows_v, al_v, dix_v, sem,
                       out_acc):
    c = lax.axis_index("c")
    sid = lax.axis_index("s")
    base = sid * EPT2

    pltpu.sync_copy(srce.at[pl.ds(base, EPT2)], src_v)
    pltpu.sync_copy(dste.at[pl.ds(base, EPT2)], dst_v)
    pltpu.sync_copy(s, s_v)
    pltpu.sync_copy(t, t_v)
    pltpu.sync_copy(dpart.at[0], den_v)
    pltpu.sync_copy(dpart.at[1], tmp_v)

    def _mrg(j, carry):
        for f in range(D // 16):
            sl = pl.ds(f * 16, 16)
            den_v[j, sl] = den_v[j, sl] + tmp_v[j, sl]
        return carry

    lax.fori_loop(0, NPR, _mrg, 0)

    hoff = c * N

    def _shift(j, carry):
        sl = pl.ds(j * 16, 16)
        src_v[sl] = src_v[sl] + hoff
        return carry

    lax.fori_loop(0, EPT2 // 16, _shift, 0)

    zero16 = jnp.zeros((16,), jnp.float32)

    def _zrows(j, carry):
        for f in range(HD // 16):
            rows_v[j, pl.ds(f * 16, 16)] = zero16
        return carry

    lax.fori_loop(0, B, _zrows, 0)
    for k in range(SLICE // B):
        pltpu.sync_copy(rows_v, out_acc.at[pl.ds(sid * SLICE + k * B, B)])
    plsc.subcore_barrier()

    def _batch(b, carry):
        e0 = b * B

        def _alpha(k, carry2):
            i16 = pl.ds(e0 + k * 16, 16)
            s16 = src_v[i16] - hoff
            d16 = dst_v[i16]
            dix_v[pl.ds(k * 16, 16)] = d16
            ev = plsc.load_gather(s_v, [s16]) + plsc.load_gather(t_v, [d16])
            ev = jnp.where(ev > 0.0, ev, 0.2 * ev)
            exv = jnp.exp(ev)
            row16 = lax.shift_right_logical(d16, 7)
            col16 = lax.bitwise_and(d16, 127)
            dnm = plsc.load_gather(den_v, [row16, col16])
            al_v[pl.ds(k * 16, 16)] = exv / (dnm + 1e-16)
            return carry2

        lax.fori_loop(0, B // 16, _alpha, 0)

        pltpu.async_copy(hcat.at[src_v.at[pl.ds(e0, B)]], rows_v, sem).wait()

        def _scale(j, carry2):
            asp = plsc.load_gather(
                al_v, [jnp.broadcast_to(j, (16,)).astype(jnp.int32)])
            for f in range(HD // 16):
                sl = pl.ds(f * 16, 16)
                rows_v[j, sl] = rows_v[j, sl] * asp
            return carry2

        lax.fori_loop(0, B, _scale, 0)

        pltpu.sync_copy(rows_v, out_acc.at[dix_v], add=True)
        return carry

    lax.fori_loop(0, NB2, _batch, 0)

    plsc.subcore_barrier()
    pltpu.sync_copy(out_acc.at[pl.ds(sid * SLICE, SLICE)],
                    opart.at[c, pl.ds(sid * SLICE, SLICE)])


def _acat(W, a_src, a_dst):
    a = jnp.zeros((D, 8), jnp.float32)
    return a.at[:, 0].set(W @ a_src).at[:, 1].set(W @ a_dst)


def _sc_layer(srce, dste, h, st):
    s, t = st[:, 0], st[:, 1]
    dp = _sc_edge_pass_kernel()(srce, dste, s, t)
    hcat = jnp.concatenate([h[:, :HD], h[:, HD:]], axis=0)
    op = _sc_aggregate_kernel()(srce, dste, s, t, dp, hcat)
    return op[0, :N], op[1, :N]


def kernel(x, edge_index, batch, W1, a_src1, a_dst1, b1, W2, a_src2, a_dst2, b2):
    srce = edge_index[0]
    dste = edge_index[1]

    h1, st1 = _tc_layer1(x, W1, _acat(W1, a_src1, a_dst1))
    p0, p1 = _sc_layer(srce, dste, h1, st1)

    h2, st2 = _tc_layer2(p0, p1, b1.reshape(1, D), W2,
                         _acat(W2, a_src2, a_dst2))
    q0, q1 = _sc_layer(srce, dste, h2, st2)

    batch3d = batch.reshape(N // _MM_BLK, 1, _MM_BLK)
    return _tc_pool(q0, q1, b2.reshape(1, D), batch3d)


# speedup vs baseline: 20.7378x; 1.5568x over previous
import functools

import jax
import jax.numpy as jnp
from jax import lax
from jax.experimental import pallas as pl
from jax.experimental.pallas import tpu as pltpu
from jax.experimental.pallas import tpu_sc as plsc

N = 10000
E = 320000
D = 128
HD = D // 2
G = 64

NC = 2
NS = 16
NP = 10240
NPR = NP // D
EPT = E // (NC * NS)
EPT2 = E // NS
B = 80
NB2 = EPT2 // B
SLICE = NP // NS


@functools.cache
def _mesh():
    return plsc.VectorSubcoreMesh(core_axis_name="c", subcore_axis_name="s",
                                  num_cores=NC, num_subcores=NS)


def _mm1_body(x_ref, w_ref, a_ref, h_ref, st_ref):
    xv = x_ref[...]
    h_ref[...] = jnp.dot(xv, w_ref[...], preferred_element_type=jnp.float32)
    st_ref[...] = jnp.dot(xv, a_ref[...], preferred_element_type=jnp.float32)


def _mm2_body(p0_ref, p1_ref, b_ref, w_ref, a_ref, h_ref, st_ref):
    xv = jnp.concatenate([p0_ref[...], p1_ref[...]], axis=1)
    xv = jnp.maximum(xv + b_ref[...], 0.0)
    h_ref[...] = jnp.dot(xv, w_ref[...], preferred_element_type=jnp.float32)
    st_ref[...] = jnp.dot(xv, a_ref[...], preferred_element_type=jnp.float32)


_MM_BLK = 2000


def _tc_layer1(x, W, acat):
    return pl.pallas_call(
        _mm1_body,
        grid=(N // _MM_BLK,),
        in_specs=[
            pl.BlockSpec((_MM_BLK, D), lambda i: (i, 0)),
            pl.BlockSpec((D, D), lambda i: (0, 0)),
            pl.BlockSpec((D, 8), lambda i: (0, 0)),
        ],
        out_specs=[
            pl.BlockSpec((_MM_BLK, D), lambda i: (i, 0)),
            pl.BlockSpec((_MM_BLK, 8), lambda i: (i, 0)),
        ],
        out_shape=[
            jax.ShapeDtypeStruct((N, D), jnp.float32),
            jax.ShapeDtypeStruct((N, 8), jnp.float32),
        ],
    )(x, W, acat)


def _tc_layer2(p0, p1, b, W, acat):
    return pl.pallas_call(
        _mm2_body,
        grid=(N // _MM_BLK,),
        in_specs=[
            pl.BlockSpec((_MM_BLK, HD), lambda i: (i, 0)),
            pl.BlockSpec((_MM_BLK, HD), lambda i: (i, 0)),
            pl.BlockSpec((1, D), lambda i: (0, 0)),
            pl.BlockSpec((D, D), lambda i: (0, 0)),
            pl.BlockSpec((D, 8), lambda i: (0, 0)),
        ],
        out_specs=[
            pl.BlockSpec((_MM_BLK, D), lambda i: (i, 0)),
            pl.BlockSpec((_MM_BLK, 8), lambda i: (i, 0)),
        ],
        out_shape=[
            jax.ShapeDtypeStruct((N, D), jnp.float32),
            jax.ShapeDtypeStruct((N, 8), jnp.float32),
        ],
    )(p0, p1, b, W, acat)


def _pool_body(p0_ref, p1_ref, b_ref, batch_ref, sum_ref, cnt_ref):
    i = pl.program_id(0)
    xv = jnp.concatenate([p0_ref[...], p1_ref[...]], axis=1)
    xv = jnp.maximum(xv + b_ref[...], 0.0)
    bt = batch_ref[...].reshape(1, _MM_BLK)
    oh = (lax.broadcasted_iota(jnp.int32, (G, _MM_BLK), 0) == bt)
    oh = oh.astype(jnp.float32)
    psum = jnp.dot(oh, xv, preferred_element_type=jnp.float32)
    pcnt = jnp.sum(oh, axis=1, keepdims=True)

    @pl.when(i == 0)
    def _():
        sum_ref[...] = jnp.zeros_like(sum_ref)
        cnt_ref[...] = jnp.zeros_like(cnt_ref)

    sum_ref[...] += psum
    cnt_ref[...] += jnp.broadcast_to(pcnt, (G, D))

    @pl.when(i == pl.num_programs(0) - 1)
    def _():
        sum_ref[...] = sum_ref[...] / jnp.maximum(cnt_ref[...], 1.0)


def _tc_pool(p0, p1, b, batch3d):
    out, _ = pl.pallas_call(
        _pool_body,
        grid=(N // _MM_BLK,),
        in_specs=[
            pl.BlockSpec((_MM_BLK, HD), lambda i: (i, 0)),
            pl.BlockSpec((_MM_BLK, HD), lambda i: (i, 0)),
            pl.BlockSpec((1, D), lambda i: (0, 0)),
            pl.BlockSpec((1, 1, _MM_BLK), lambda i: (i, 0, 0)),
        ],
        out_specs=[
            pl.BlockSpec((G, D), lambda i: (0, 0)),
            pl.BlockSpec((G, D), lambda i: (0, 0)),
        ],
        out_shape=[
            jax.ShapeDtypeStruct((G, D), jnp.float32),
            jax.ShapeDtypeStruct((G, D), jnp.float32),
        ],
    )(p0, p1, b, batch3d)
    return out


def _dmerge_body(dp_ref, o_ref):
    o_ref[...] = dp_ref[0] + dp_ref[1]


def _tc_dmerge(dp):
    return pl.pallas_call(
        _dmerge_body,
        out_shape=jax.ShapeDtypeStruct((NPR, D), jnp.float32),
    )(dp)


@functools.cache
def _sc_edge_pass_kernel():
  return functools.partial(
    pl.kernel,
    out_type=jax.ShapeDtypeStruct((NC, NPR, D), jnp.float32),
    mesh=_mesh(),
    scratch_types=[
        pltpu.VMEM((EPT,), jnp.int32),
        pltpu.VMEM((EPT,), jnp.int32),
        pltpu.VMEM((N,), jnp.float32),
        pltpu.VMEM((N,), jnp.float32),
        pltpu.VMEM((NPR, D), jnp.float32),
        pltpu.VMEM((NPR,), jnp.int32),
        pltpu.VMEM((5, D), jnp.float32),
        pltpu.VMEM_SHARED((NPR, D), jnp.float32),
    ],
    compiler_params=pltpu.CompilerParams(needs_layout_passes=False),
  )(_sc_edge_pass_body)


def _sc_edge_pass_body(psd, s, t, dpart, src_v, dst_v, s_v, t_v,
                       den_v, rix_v, z5_v, stage):
    c = lax.axis_index("c")
    sid = lax.axis_index("s")
    base = (c * NS + sid) * EPT

    pltpu.sync_copy(psd.at[pl.ds(base, EPT)], src_v)
    pltpu.sync_copy(s, s_v)
    pltpu.sync_copy(t, t_v)

    def _unpack(j, carry):
        sl = pl.ds(j * 16, 16)
        v = src_v[sl]
        dst_v[sl] = lax.shift_right_logical(v, 14)
        src_v[sl] = lax.bitwise_and(v, 16383)
        return carry

    lax.fori_loop(0, EPT // 16, _unpack, 0)

    zero16 = jnp.zeros((16,), jnp.float32)
    for r in range(5):
        for f in range(D // 16):
            z5_v[r, pl.ds(f * 16, 16)] = zero16

    def _zero_den(j, carry):
        for f in range(D // 16):
            den_v[j, pl.ds(f * 16, 16)] = zero16
        return carry

    lax.fori_loop(0, NPR, _zero_den, 0)

    for k in range(NPR // 16):
        rix_v[pl.ds(k * 16, 16)] = lax.iota(jnp.int32, 16) + k * 16

    pltpu.sync_copy(z5_v, stage.at[pl.ds(sid * 5, 5)])

    def _edge(j, carry):
        s16 = src_v[pl.ds(j * 16, 16)]
        d16 = dst_v[pl.ds(j * 16, 16)]
        ev = plsc.load_gather(s_v, [s16]) + plsc.load_gather(t_v, [d16])
        ev = jnp.where(ev > 0.0, ev, 0.2 * ev)
        exv = jnp.exp(ev)
        row16 = lax.shift_right_logical(d16, 7)
        col16 = lax.bitwise_and(d16, 127)
        plsc.addupdate_scatter(den_v, [row16, col16], exv)
        return carry

    lax.fori_loop(0, EPT // 16, _edge, 0)

    plsc.subcore_barrier()
    pltpu.sync_copy(den_v, stage.at[rix_v], add=True)
    plsc.subcore_barrier()

    @pl.when(sid < NPR // 8)
    def _():
        pltpu.sync_copy(stage.at[pl.ds(sid * 8, 8)],
                        dpart.at[c, pl.ds(sid * 8, 8)])


@functools.cache
def _sc_aggregate_kernel():
  return functools.partial(
    pl.kernel,
    out_type=jax.ShapeDtypeStruct((NC, NP, HD), jnp.float32),
    mesh=_mesh(),
    scratch_types=[
        pltpu.VMEM((EPT2,), jnp.int32),
        pltpu.VMEM((EPT2,), jnp.int32),
        pltpu.VMEM((N,), jnp.float32),
        pltpu.VMEM((N,), jnp.float32),
        pltpu.VMEM((NPR, D), jnp.float32),
        pltpu.VMEM((B, HD), jnp.float32),
        pltpu.VMEM((B, HD), jnp.float32),
        pltpu.VMEM((B,), jnp.float32),
        pltpu.VMEM((B,), jnp.int32),
        pltpu.SemaphoreType.DMA,
        pltpu.VMEM_SHARED((N, HD), jnp.float32),
    ],
    compiler_params=pltpu.CompilerParams(needs_layout_passes=False,
                                         use_tc_tiling_on_sc=False),
  )(_sc_aggregate_body)


def _sc_aggregate_body(psd, s, t, den, hcat, opart, src_v, dst_v,
                       s_v, t_v, den_v, rows0_v, rows1_v, al_v,
                       dix_v, semg, out_acc):
    rows_b = (rows0_v, rows1_v)
    c = lax.axis_index("c")
    sid = lax.axis_index("s")
    base = sid * EPT2

    pltpu.sync_copy(psd.at[pl.ds(base, EPT2)], src_v)
    pltpu.sync_copy(s, s_v)
    pltpu.sync_copy(t, t_v)
    pltpu.sync_copy(den, den_v)

    hoff = c * N

    def _unpack(j, carry):
        sl = pl.ds(j * 16, 16)
        v = src_v[sl]
        dst_v[sl] = lax.shift_right_logical(v, 14)
        src_v[sl] = lax.bitwise_and(v, 16383) + hoff
        return carry

    lax.fori_loop(0, EPT2 // 16, _unpack, 0)

    zero16 = jnp.zeros((16,), jnp.float32)

    def _zrows(j, carry):
        for f in range(HD // 16):
            rows0_v[j, pl.ds(f * 16, 16)] = zero16
        return carry

    lax.fori_loop(0, B, _zrows, 0)
    for k in range(7):
        pltpu.sync_copy(rows0_v, out_acc.at[pl.ds(sid * 625 + k * B, B)])
    pltpu.sync_copy(rows0_v.at[pl.ds(0, 65)],
                    out_acc.at[pl.ds(sid * 625 + 560, 65)])
    plsc.subcore_barrier()

    pltpu.async_copy(hcat.at[src_v.at[pl.ds(0, B)]], rows0_v, semg).wait()

    def _process(b, cur):
        rows = rows_b[cur]
        orows = rows_b[1 - cur]

        nxt = jnp.minimum(b + 1, NB2 - 1)
        gd = pltpu.async_copy(hcat.at[src_v.at[pl.ds(nxt * B, B)]], orows,
                              semg)

        e0 = b * B

        def _alpha(k, carry2):
            i16 = pl.ds(e0 + k * 16, 16)
            s16 = src_v[i16] - hoff
            d16 = dst_v[i16]
            dix_v[pl.ds(k * 16, 16)] = d16
            ev = plsc.load_gather(s_v, [s16]) + plsc.load_gather(t_v, [d16])
            ev = jnp.where(ev > 0.0, ev, 0.2 * ev)
            exv = jnp.exp(ev)
            row16 = lax.shift_right_logical(d16, 7)
            col16 = lax.bitwise_and(d16, 127)
            dnm = plsc.load_gather(den_v, [row16, col16])
            al_v[pl.ds(k * 16, 16)] = exv / (dnm + 1e-16)
            return carry2

        lax.fori_loop(0, B // 16, _alpha, 0)

        def _scale(j, carry2):
            asp = plsc.load_gather(
                al_v, [jnp.broadcast_to(j, (16,)).astype(jnp.int32)])
            for f in range(HD // 16):
                sl = pl.ds(f * 16, 16)
                rows[j, sl] = rows[j, sl] * asp
            return carry2

        lax.fori_loop(0, B, _scale, 0)

        pltpu.sync_copy(rows, out_acc.at[dix_v], add=True)
        gd.wait()

    def _pair(i, carry):
        _process(2 * i, 0)
        _process(2 * i + 1, 1)
        return carry

    lax.fori_loop(0, NB2 // 2, _pair, 0)

    plsc.subcore_barrier()
    pltpu.sync_copy(out_acc.at[pl.ds(sid * 625, 625)],
                    opart.at[c, pl.ds(sid * 625, 625)])


def _acat(W, a_src, a_dst):
    a = jnp.zeros((D, 8), jnp.float32)
    return a.at[:, 0].set(W @ a_src).at[:, 1].set(W @ a_dst)


def _sc_layer(psd, h, st):
    s, t = st[:, 0], st[:, 1]
    dp = _sc_edge_pass_kernel()(psd, s, t)
    den = _tc_dmerge(dp)
    hcat = jnp.concatenate([h[:, :HD], h[:, HD:]], axis=0)
    op = _sc_aggregate_kernel()(psd, s, t, den, hcat)
    return op[0, :N], op[1, :N]


def kernel(x, edge_index, batch, W1, a_src1, a_dst1, b1, W2, a_src2, a_dst2, b2):
    psd = edge_index[0] + edge_index[1] * 16384

    h1, st1 = _tc_layer1(x, W1, _acat(W1, a_src1, a_dst1))
    p0, p1 = _sc_layer(psd, h1, st1)

    h2, st2 = _tc_layer2(p0, p1, b1.reshape(1, D), W2,
                         _acat(W2, a_src2, a_dst2))
    q0, q1 = _sc_layer(psd, h2, st2)

    batch3d = batch.reshape(N // _MM_BLK, 1, _MM_BLK)
    return _tc_pool(q0, q1, b2.reshape(1, D), batch3d)


# speedup vs baseline: 24.4322x; 1.1781x over previous
import functools

import jax
import jax.numpy as jnp
from jax import lax
from jax.experimental import pallas as pl
from jax.experimental.pallas import tpu as pltpu
from jax.experimental.pallas import tpu_sc as plsc

N = 10000
E = 320000
D = 128
HD = D // 2
G = 64

NC = 2
NS = 16
NP = 10240
NPR = NP // D
EPT = E // (NC * NS)
EPT2 = E // NS
B = 80
NB2 = EPT2 // B
SLICE = NP // NS


@functools.cache
def _mesh():
    return plsc.VectorSubcoreMesh(core_axis_name="c", subcore_axis_name="s",
                                  num_cores=NC, num_subcores=NS)


def _mm1_body(x_ref, w_ref, a_ref, h_ref, st_ref):
    xv = x_ref[...]
    h_ref[...] = jnp.dot(xv, w_ref[...], preferred_element_type=jnp.float32)
    st_ref[...] = jnp.dot(xv, a_ref[...], preferred_element_type=jnp.float32)


def _mm2_body(p0_ref, p1_ref, b_ref, w_ref, a_ref, h_ref, st_ref):
    xv = jnp.concatenate([p0_ref[...], p1_ref[...]], axis=1)
    xv = jnp.maximum(xv + b_ref[...], 0.0)
    h_ref[...] = jnp.dot(xv, w_ref[...], preferred_element_type=jnp.float32)
    st_ref[...] = jnp.dot(xv, a_ref[...], preferred_element_type=jnp.float32)


_MM_BLK = 2000


def _tc_layer1(x, W, acat):
    return pl.pallas_call(
        _mm1_body,
        grid=(N // _MM_BLK,),
        in_specs=[
            pl.BlockSpec((_MM_BLK, D), lambda i: (i, 0)),
            pl.BlockSpec((D, D), lambda i: (0, 0)),
            pl.BlockSpec((D, 8), lambda i: (0, 0)),
        ],
        out_specs=[
            pl.BlockSpec((_MM_BLK, D), lambda i: (i, 0)),
            pl.BlockSpec((_MM_BLK, 8), lambda i: (i, 0)),
        ],
        out_shape=[
            jax.ShapeDtypeStruct((N, D), jnp.float32),
            jax.ShapeDtypeStruct((N, 8), jnp.float32),
        ],
    )(x, W, acat)


def _tc_layer2(p0, p1, b, W, acat):
    return pl.pallas_call(
        _mm2_body,
        grid=(N // _MM_BLK,),
        in_specs=[
            pl.BlockSpec((_MM_BLK, HD), lambda i: (i, 0)),
            pl.BlockSpec((_MM_BLK, HD), lambda i: (i, 0)),
            pl.BlockSpec((1, D), lambda i: (0, 0)),
            pl.BlockSpec((D, D), lambda i: (0, 0)),
            pl.BlockSpec((D, 8), lambda i: (0, 0)),
        ],
        out_specs=[
            pl.BlockSpec((_MM_BLK, D), lambda i: (i, 0)),
            pl.BlockSpec((_MM_BLK, 8), lambda i: (i, 0)),
        ],
        out_shape=[
            jax.ShapeDtypeStruct((N, D), jnp.float32),
            jax.ShapeDtypeStruct((N, 8), jnp.float32),
        ],
    )(p0, p1, b, W, acat)


def _pool_body(p0_ref, p1_ref, b_ref, batch_ref, sum_ref, cnt_ref):
    i = pl.program_id(0)
    xv = jnp.concatenate([p0_ref[...], p1_ref[...]], axis=1)
    xv = jnp.maximum(xv + b_ref[...], 0.0)
    bt = batch_ref[...].reshape(1, _MM_BLK)
    oh = (lax.broadcasted_iota(jnp.int32, (G, _MM_BLK), 0) == bt)
    oh = oh.astype(jnp.float32)
    psum = jnp.dot(oh, xv, preferred_element_type=jnp.float32)
    pcnt = jnp.sum(oh, axis=1, keepdims=True)

    @pl.when(i == 0)
    def _():
        sum_ref[...] = jnp.zeros_like(sum_ref)
        cnt_ref[...] = jnp.zeros_like(cnt_ref)

    sum_ref[...] += psum
    cnt_ref[...] += jnp.broadcast_to(pcnt, (G, D))

    @pl.when(i == pl.num_programs(0) - 1)
    def _():
        sum_ref[...] = sum_ref[...] / jnp.maximum(cnt_ref[...], 1.0)


def _tc_pool(p0, p1, b, batch3d):
    out, _ = pl.pallas_call(
        _pool_body,
        grid=(N // _MM_BLK,),
        in_specs=[
            pl.BlockSpec((_MM_BLK, HD), lambda i: (i, 0)),
            pl.BlockSpec((_MM_BLK, HD), lambda i: (i, 0)),
            pl.BlockSpec((1, D), lambda i: (0, 0)),
            pl.BlockSpec((1, 1, _MM_BLK), lambda i: (i, 0, 0)),
        ],
        out_specs=[
            pl.BlockSpec((G, D), lambda i: (0, 0)),
            pl.BlockSpec((G, D), lambda i: (0, 0)),
        ],
        out_shape=[
            jax.ShapeDtypeStruct((G, D), jnp.float32),
            jax.ShapeDtypeStruct((G, D), jnp.float32),
        ],
    )(p0, p1, b, batch3d)
    return out


@functools.cache
def _sc_layer_kernel():
  return functools.partial(
    pl.kernel,
    out_type=jax.ShapeDtypeStruct((NC, NP, HD), jnp.float32),
    mesh=_mesh(),
    scratch_types=[
        pltpu.VMEM((EPT2,), jnp.int32),
        pltpu.VMEM((EPT2,), jnp.int32),
        pltpu.VMEM((N,), jnp.float32),
        pltpu.VMEM((N,), jnp.float32),
        pltpu.VMEM((NPR, D), jnp.float32),
        pltpu.VMEM((NPR,), jnp.int32),
        pltpu.VMEM((5, D), jnp.float32),
        pltpu.VMEM((B, HD), jnp.float32),
        pltpu.VMEM((B, HD), jnp.float32),
        pltpu.VMEM((B,), jnp.float32),
        pltpu.VMEM((B,), jnp.int32),
        pltpu.VMEM((B,), jnp.int32),
        pltpu.SemaphoreType.DMA,
        pltpu.SemaphoreType.DMA,
        pltpu.SemaphoreType.DMA,
        pltpu.VMEM_SHARED((NPR, D), jnp.float32),
        pltpu.VMEM_SHARED((N, HD), jnp.float32),
    ],
    compiler_params=pltpu.CompilerParams(needs_layout_passes=False,
                                         use_tc_tiling_on_sc=False),
  )(_sc_layer_body)


def _sc_layer_body(psd, s, t, hcat, opart, src_v, dst_v, s_v, t_v, den_v,
                   rix_v, z5_v, rows0_v, rows1_v, al_v, dix0_v, dix1_v,
                   semg, sems0, sems1, stage, out_acc):
    rows_b = (rows0_v, rows1_v)
    dix_b = (dix0_v, dix1_v)
    sems_b = (sems0, sems1)
    c = lax.axis_index("c")
    sid = lax.axis_index("s")
    base = sid * EPT2

    pltpu.sync_copy(psd.at[pl.ds(base, EPT2)], src_v)
    pltpu.sync_copy(s, s_v)
    pltpu.sync_copy(t, t_v)

    hoff = c * N

    @plsc.parallel_loop(0, EPT2 // 16, unroll=8)
    def _unpack(j):
        sl = pl.ds(j * 16, 16)
        v = src_v[sl]
        dst_v[sl] = lax.shift_right_logical(v, 14)
        src_v[sl] = lax.bitwise_and(v, 16383) + hoff

    zero16 = jnp.zeros((16,), jnp.float32)
    for r in range(5):
        for f in range(D // 16):
            z5_v[r, pl.ds(f * 16, 16)] = zero16

    @plsc.parallel_loop(0, NPR, unroll=8)
    def _zero_den(j):
        for f in range(D // 16):
            den_v[j, pl.ds(f * 16, 16)] = zero16

    for k in range(NPR // 16):
        rix_v[pl.ds(k * 16, 16)] = lax.iota(jnp.int32, 16) + k * 16

    pltpu.sync_copy(z5_v, stage.at[pl.ds(sid * 5, 5)])

    @plsc.parallel_loop(0, EPT2 // 16, unroll=8)
    def _edge(j):
        s16 = src_v[pl.ds(j * 16, 16)] - hoff
        d16 = dst_v[pl.ds(j * 16, 16)]
        ev = plsc.load_gather(s_v, [s16]) + plsc.load_gather(t_v, [d16])
        ev = jnp.where(ev > 0.0, ev, 0.2 * ev)
        exv = jnp.exp(ev)
        row16 = lax.shift_right_logical(d16, 7)
        col16 = lax.bitwise_and(d16, 127)
        plsc.addupdate_scatter(den_v, [row16, col16], exv)

    @plsc.parallel_loop(0, B, unroll=8)
    def _zrows(j):
        for f in range(HD // 16):
            rows0_v[j, pl.ds(f * 16, 16)] = zero16
    for k in range(7):
        pltpu.sync_copy(rows0_v, out_acc.at[pl.ds(sid * 625 + k * B, B)])
    pltpu.sync_copy(rows0_v.at[pl.ds(0, 65)],
                    out_acc.at[pl.ds(sid * 625 + 560, 65)])

    plsc.subcore_barrier()
    pltpu.sync_copy(den_v, stage.at[rix_v], add=True)
    plsc.subcore_barrier()
    pltpu.sync_copy(stage, den_v)

    pltpu.async_copy(hcat.at[src_v.at[pl.ds(0, B)]], rows0_v, semg).wait()

    def _process(b, cur, prev_sd):
        rows = rows_b[cur]
        orows = rows_b[1 - cur]
        dix = dix_b[cur]

        if prev_sd is not None:
            prev_sd.wait()
        nxt = jnp.minimum(b + 1, NB2 - 1)
        gd = pltpu.async_copy(hcat.at[src_v.at[pl.ds(nxt * B, B)]], orows,
                              semg)

        e0 = b * B

        @plsc.parallel_loop(0, B // 16, unroll=5)
        def _alpha(k):
            i16 = pl.ds(e0 + k * 16, 16)
            s16 = src_v[i16] - hoff
            d16 = dst_v[i16]
            dix[pl.ds(k * 16, 16)] = d16
            ev = plsc.load_gather(s_v, [s16]) + plsc.load_gather(t_v, [d16])
            ev = jnp.where(ev > 0.0, ev, 0.2 * ev)
            exv = jnp.exp(ev)
            row16 = lax.shift_right_logical(d16, 7)
            col16 = lax.bitwise_and(d16, 127)
            dnm = plsc.load_gather(den_v, [row16, col16])
            al_v[pl.ds(k * 16, 16)] = exv / (dnm + 1e-16)

        @plsc.parallel_loop(0, B, unroll=8)
        def _scale(j):
            asp = plsc.load_gather(
                al_v, [jnp.broadcast_to(j, (16,)).astype(jnp.int32)])
            for f in range(HD // 16):
                sl = pl.ds(f * 16, 16)
                rows[j, sl] = rows[j, sl] * asp

        sd = pltpu.async_copy(rows, out_acc.at[dix], sems_b[cur], add=True)
        gd.wait()
        return sd

    def _pair(i, carry):
        sd0 = _process(2 * i, 0, None)
        sd1 = _process(2 * i + 1, 1, sd0)
        sd1.wait()
        return carry

    lax.fori_loop(0, NB2 // 2, _pair, 0)

    plsc.subcore_barrier()
    pltpu.sync_copy(out_acc.at[pl.ds(sid * 625, 625)],
                    opart.at[c, pl.ds(sid * 625, 625)])


def _acat(W, a_src, a_dst):
    a = jnp.zeros((D, 8), jnp.float32)
    return a.at[:, 0].set(W @ a_src).at[:, 1].set(W @ a_dst)


def _sc_layer(psd, h, st):
    s, t = st[:, 0], st[:, 1]
    hcat = jnp.concatenate([h[:, :HD], h[:, HD:]], axis=0)
    op = _sc_layer_kernel()(psd, s, t, hcat)
    return op[0, :N], op[1, :N]


def kernel(x, edge_index, batch, W1, a_src1, a_dst1, b1, W2, a_src2, a_dst2, b2):
    psd = edge_index[0] + edge_index[1] * 16384

    h1, st1 = _tc_layer1(x, W1, _acat(W1, a_src1, a_dst1))
    p0, p1 = _sc_layer(psd, h1, st1)

    h2, st2 = _tc_layer2(p0, p1, b1.reshape(1, D), W2,
                         _acat(W2, a_src2, a_dst2))
    q0, q1 = _sc_layer(psd, h2, st2)

    batch3d = batch.reshape(N // _MM_BLK, 1, _MM_BLK)
    return _tc_pool(q0, q1, b2.reshape(1, D), batch3d)
